# baseline (speedup 1.0000x reference)
import functools

import jax
import jax.numpy as jnp
from jax import lax
from jax.experimental import pallas as pl
from jax.experimental.pallas import tpu as pltpu
from jax.experimental.pallas import tpu_sc as plsc

N_NODES = 10000
F = 128
R_REL = 8
DEG = 32
L = 16
C = 8
EDGES_PER_CHUNK = C * DEG
NUM_CHUNKS = N_NODES // C
NW = 32


def _tc_transform(x, linear):
    BN = 10000
    nb = N_NODES // BN

    def body(x_ref, w_ref, o_ref):
        o_ref[...] = jnp.dot(x_ref[...], w_ref[0],
                             preferred_element_type=jnp.float32)

    return pl.pallas_call(
        body,
        grid=(nb, R_REL),
        in_specs=[
            pl.BlockSpec((BN, F), lambda j, r: (j, 0)),
            pl.BlockSpec((1, F, F), lambda j, r: (r, 0, 0)),
        ],
        out_specs=pl.BlockSpec((BN, F), lambda j, r: (r * nb + j, 0)),
        out_shape=jax.ShapeDtypeStruct((R_REL * N_NODES, F), jnp.float32),
        compiler_params=pltpu.CompilerParams(
            dimension_semantics=("parallel", "arbitrary")),
    )(x, linear)


def _sc_gather_mean(xt, idx, et, hm, hb):
    mesh = plsc.VectorSubcoreMesh(core_axis_name="c", subcore_axis_name="s")
    T_CH = (NUM_CHUNKS + NW - 1) // NW
    assert T_CH % 2 == 0

    @functools.partial(
        pl.kernel,
        mesh=mesh,
        out_type=jax.ShapeDtypeStruct((N_NODES, F), jnp.float32),
        scratch_types=[
            pltpu.VMEM((2, EDGES_PER_CHUNK), jnp.int32),
            pltpu.VMEM((2, EDGES_PER_CHUNK), jnp.int32),
            pltpu.VMEM((2, 4, 64), jnp.int32),
            pltpu.VMEM((2, 4, 64, F), jnp.float32),
            pltpu.VMEM((2, L), jnp.int32),
            pltpu.VMEM((2, C, F), jnp.float32),
            pltpu.VMEM((2, C, F), jnp.float32),
            pltpu.SemaphoreType.DMA,
            pltpu.SemaphoreType.DMA,
            pltpu.SemaphoreType.DMA,
            pltpu.SemaphoreType.DMA,
            pltpu.SemaphoreType.DMA,
            pltpu.SemaphoreType.DMA,
        ],
    )
    def k(xt_hbm, idx_hbm, et_hbm, hm_hbm, hb_hbm, out_hbm,
          idxv, etv, combv, rowsv, hmv, hbv, outv,
          sem_ie0, sem_ie1, sem_g0, sem_g1, sem_o0, sem_o1):
        cid = lax.axis_index("c")
        sid = lax.axis_index("s")
        wid = sid * 2 + cid
        sem_ie = (sem_ie0, sem_ie1)
        sem_g = (sem_g0, sem_g1)
        sem_o = (sem_o0, sem_o1)

        def valid(t):
            return (t >= 0) & (t * NW + wid < NUM_CHUNKS)

        def ie_descs(t, a):
            ebase = (t * NW + wid) * EDGES_PER_CHUNK
            return [
                pltpu.make_async_copy(
                    idx_hbm.at[pl.ds(ebase, EDGES_PER_CHUNK)], idxv.at[a],
                    sem_ie[a]),
                pltpu.make_async_copy(
                    et_hbm.at[pl.ds(ebase, EDGES_PER_CHUNK)], etv.at[a],
                    sem_ie[a]),
            ]

        def gather_descs(t, a):
            nbase = (t * NW + wid) * C
            return [
                pltpu.make_async_copy(
                    xt_hbm.at[combv.at[a, j]], rowsv.at[a, j], sem_g[a])
                for j in range(4)
            ] + [
                pltpu.make_async_copy(
                    hm_hbm.at[pl.ds(nbase, C)], hmv.at[a, pl.ds(0, C)],
                    sem_g[a]),
                pltpu.make_async_copy(
                    hb_hbm.at[pl.ds(nbase, C)], hbv.at[a], sem_g[a]),
            ]

        def out_desc(t, a):
            nbase = (t * NW + wid) * C
            return pltpu.make_async_copy(
                outv.at[a], out_hbm.at[pl.ds(nbase, C)], sem_o[a])

        def reduce_chunk(a):
            hm16 = hmv[a]
            for c in range(C):
                b = (c * DEG) // 64
                r0 = (c * DEG) % 64
                use_hist = hm16[c] != jnp.int32(-1)

                def row_red(r, accs, _b=b, _r0=r0):
                    return tuple(
                        accs[kk] + rowsv[a, _b, _r0 + r, pl.ds(kk * L, L)]
                        for kk in range(F // L))

                accs = lax.fori_loop(
                    0, DEG, row_red,
                    tuple(jnp.zeros((L,), jnp.float32) for _ in range(F // L)),
                    unroll=8)
                for kk in range(F // L):
                    val = jnp.where(use_hist, hbv[a, c, pl.ds(kk * L, L)],
                                    accs[kk] * jnp.float32(1.0 / DEG))
                    outv[a, c, pl.ds(kk * L, L)] = val

        def chunk_step(t, a):
            bb = 1 - a

            @pl.when(valid(t))
            def _():
                for d in ie_descs(t, a):
                    d.wait()
                for i in range(EDGES_PER_CHUNK // L):
                    combv[a, i // 4, pl.ds((i % 4) * L, L)] = (
                        etv[a, pl.ds(i * L, L)] * N_NODES
                        + idxv[a, pl.ds(i * L, L)])
                for d in gather_descs(t, a):
                    d.start()

            @pl.when(valid(t + 1))
            def _():
                for d in ie_descs(t + 1, bb):
                    d.start()

            @pl.when(valid(t - 3))
            def _():
                out_desc(t - 3, bb).wait()

            @pl.when(valid(t - 1))
            def _():
                for d in gather_descs(t - 1, bb):
                    d.wait()
                reduce_chunk(bb)
                out_desc(t - 1, bb).start()

        for d in ie_descs(0, 0):
            d.start()

        def pair_body(u, carry):
            chunk_step(2 * u, 0)
            chunk_step(2 * u + 1, 1)
            return carry

        lax.fori_loop(0, T_CH // 2, pair_body, 0)

        tl = T_CH - 1

        @pl.when(valid(tl - 2))
        def _():
            out_desc(tl - 2, 1).wait()

        @pl.when(valid(tl))
        def _():
            for d in gather_descs(tl, 1):
                d.wait()
            reduce_chunk(1)
            out_desc(tl, 1).start()

        @pl.when(valid(tl - 1))
        def _():
            out_desc(tl - 1, 0).wait()

        @pl.when(valid(tl))
        def _():
            out_desc(tl, 1).wait()

    return k(xt, idx, et, hm, hb)


def kernel(x, linear, ptr, idx, edge_types, history_map, history_buffer,
           history_size, num_node):
    xt = _tc_transform(x, linear)
    hm = jnp.where(jnp.asarray(history_size) > 0, history_map, jnp.int32(-1))
    out = _sc_gather_mean(xt, idx, edge_types, hm, history_buffer)
    return (out, out)


# speedup vs baseline: 1.0087x; 1.0087x over previous
import functools

import jax
import jax.numpy as jnp
from jax import lax
from jax.experimental import pallas as pl
from jax.experimental.pallas import tpu as pltpu
from jax.experimental.pallas import tpu_sc as plsc

N_NODES = 10000
F = 128
R_REL = 8
DEG = 32
L = 16
C = 8
EDGES_PER_CHUNK = C * DEG
NUM_CHUNKS = N_NODES // C
NW = 32


def _tc_transform(x, linear):
    BN = 10000
    nb = N_NODES // BN

    def body(x_ref, w_ref, o_ref):
        o_ref[...] = jnp.dot(x_ref[...], w_ref[0],
                             preferred_element_type=jnp.float32)

    return pl.pallas_call(
        body,
        grid=(nb, R_REL),
        in_specs=[
            pl.BlockSpec((BN, F), lambda j, r: (j, 0)),
            pl.BlockSpec((1, F, F), lambda j, r: (r, 0, 0)),
        ],
        out_specs=pl.BlockSpec((BN, F), lambda j, r: (r * nb + j, 0)),
        out_shape=jax.ShapeDtypeStruct((R_REL * N_NODES, F), jnp.float32),
        compiler_params=pltpu.CompilerParams(
            dimension_semantics=("parallel", "arbitrary")),
    )(x, linear)


def _sc_gather_mean(xt, idx, et, hm, hb):
    mesh = plsc.VectorSubcoreMesh(core_axis_name="c", subcore_axis_name="s")
    T_CH = (NUM_CHUNKS + NW - 1) // NW
    assert T_CH % 2 == 0

    @functools.partial(
        pl.kernel,
        mesh=mesh,
        out_type=jax.ShapeDtypeStruct((N_NODES, F), jnp.float32),
        scratch_types=[
            pltpu.VMEM((2, EDGES_PER_CHUNK), jnp.int32),
            pltpu.VMEM((2, EDGES_PER_CHUNK), jnp.int32),
            pltpu.VMEM((2, 2, 128), jnp.int32),
            pltpu.VMEM((2, 2, 128, F), jnp.float32),
            pltpu.VMEM((2, L), jnp.int32),
            pltpu.VMEM((2, C, F), jnp.float32),
            pltpu.VMEM((2, C, F), jnp.float32),
            pltpu.SemaphoreType.DMA,
            pltpu.SemaphoreType.DMA,
            pltpu.SemaphoreType.DMA,
            pltpu.SemaphoreType.DMA,
            pltpu.SemaphoreType.DMA,
            pltpu.SemaphoreType.DMA,
        ],
    )
    def k(xt_hbm, idx_hbm, et_hbm, hm_hbm, hb_hbm, out_hbm,
          idxv, etv, combv, rowsv, hmv, hbv, outv,
          sem_ie0, sem_ie1, sem_g0, sem_g1, sem_o0, sem_o1):
        cid = lax.axis_index("c")
        sid = lax.axis_index("s")
        wid = sid * 2 + cid
        sem_ie = (sem_ie0, sem_ie1)
        sem_g = (sem_g0, sem_g1)
        sem_o = (sem_o0, sem_o1)

        def valid(t):
            return (t >= 0) & (t * NW + wid < NUM_CHUNKS)

        def ie_descs(t, a):
            ebase = (t * NW + wid) * EDGES_PER_CHUNK
            return [
                pltpu.make_async_copy(
                    idx_hbm.at[pl.ds(ebase, EDGES_PER_CHUNK)], idxv.at[a],
                    sem_ie[a]),
                pltpu.make_async_copy(
                    et_hbm.at[pl.ds(ebase, EDGES_PER_CHUNK)], etv.at[a],
                    sem_ie[a]),
            ]

        def gather_descs(t, a):
            nbase = (t * NW + wid) * C
            return [
                pltpu.make_async_copy(
                    xt_hbm.at[combv.at[a, 0]], rowsv.at[a, 0], sem_g[a]),
                pltpu.make_async_copy(
                    xt_hbm.at[combv.at[a, 1]], rowsv.at[a, 1], sem_g[a]),
                pltpu.make_async_copy(
                    hm_hbm.at[pl.ds(nbase, C)], hmv.at[a, pl.ds(0, C)],
                    sem_g[a]),
                pltpu.make_async_copy(
                    hb_hbm.at[pl.ds(nbase, C)], hbv.at[a], sem_g[a]),
            ]

        def out_desc(t, a):
            nbase = (t * NW + wid) * C
            return pltpu.make_async_copy(
                outv.at[a], out_hbm.at[pl.ds(nbase, C)], sem_o[a])

        def reduce_chunk(a):
            hm16 = hmv[a]
            for c in range(C):
                b = (c * DEG) // 128
                r0 = (c * DEG) % 128
                use_hist = hm16[c] != jnp.int32(-1)

                def row_red(r, accs, _b=b, _r0=r0):
                    return tuple(
                        accs[kk] + rowsv[a, _b, _r0 + r, pl.ds(kk * L, L)]
                        for kk in range(F // L))

                accs = lax.fori_loop(
                    0, DEG, row_red,
                    tuple(jnp.zeros((L,), jnp.float32) for _ in range(F // L)),
                    unroll=8)
                for kk in range(F // L):
                    val = jnp.where(use_hist, hbv[a, c, pl.ds(kk * L, L)],
                                    accs[kk] * jnp.float32(1.0 / DEG))
                    outv[a, c, pl.ds(kk * L, L)] = val

        def chunk_step(t, a):
            bb = 1 - a

            @pl.when(valid(t))
            def _():
                for d in ie_descs(t, a):
                    d.wait()
                for i in range(EDGES_PER_CHUNK // L):
                    combv[a, i // 8, pl.ds((i % 8) * L, L)] = (
                        etv[a, pl.ds(i * L, L)] * N_NODES
                        + idxv[a, pl.ds(i * L, L)])
                for d in gather_descs(t, a):
                    d.start()

            @pl.when(valid(t + 1))
            def _():
                for d in ie_descs(t + 1, bb):
                    d.start()

            @pl.when(valid(t - 3))
            def _():
                out_desc(t - 3, bb).wait()

            @pl.when(valid(t - 1))
            def _():
                for d in gather_descs(t - 1, bb):
                    d.wait()
                reduce_chunk(bb)
                out_desc(t - 1, bb).start()

        for d in ie_descs(0, 0):
            d.start()

        def pair_body(u, carry):
            chunk_step(2 * u, 0)
            chunk_step(2 * u + 1, 1)
            return carry

        lax.fori_loop(0, T_CH // 2, pair_body, 0)

        tl = T_CH - 1

        @pl.when(valid(tl - 2))
        def _():
            out_desc(tl - 2, 1).wait()

        @pl.when(valid(tl))
        def _():
            for d in gather_descs(tl, 1):
                d.wait()
            reduce_chunk(1)
            out_desc(tl, 1).start()

        @pl.when(valid(tl - 1))
        def _():
            out_desc(tl - 1, 0).wait()

        @pl.when(valid(tl))
        def _():
            out_desc(tl, 1).wait()

    return k(xt, idx, et, hm, hb)


def kernel(x, linear, ptr, idx, edge_types, history_map, history_buffer,
           history_size, num_node):
    xt = _tc_transform(x, linear)
    hm = jnp.where(jnp.asarray(history_size) > 0, history_map, jnp.int32(-1))
    out = _sc_gather_mean(xt, idx, edge_types, hm, history_buffer)
    return (out, out)
